# Initial kernel scaffold; baseline (speedup 1.0000x reference)
import jax
import jax.numpy as jnp
from jax.experimental import pallas as pl


def kernel(species12, distances, cutoff_radii):
    raise NotImplementedError("write your pallas kernel here")


# speedup vs baseline: 260.6595x; 260.6595x over previous
import functools

import jax
import jax.numpy as jnp
from jax import lax
from jax.experimental import pallas as pl
from jax.experimental.pallas import tpu as pltpu
from jax.experimental.pallas import tpu_sc as plsc

A1 = 0.3981
A2 = 4.4211
LANES = 16


def _pow6(x):
    x2 = x * x
    return x2 * x2 * x2


def _tec_body(chunk, n_chunks, p_total, species_hbm, dist_hbm, table_hbm,
              out_hbm, tbl_v, s1a, s1b, s2a, s2b, da, db, oa, ob, sems):
    nc = 2
    wid = lax.axis_index("s") * nc + lax.axis_index("c")
    per_worker = chunk * n_chunks
    base = wid * per_worker
    bufs = [(s1a, s2a, da, oa), (s1b, s2b, db, ob)]

    pltpu.sync_copy(table_hbm, tbl_v)
    damp_tbl = _pow6(A1 * tbl_v[...] + A2)

    def in_copies(c, b):
        off = base + c * chunk
        s1_v, s2_v, d_v, _ = bufs[b]
        return [
            pltpu.make_async_copy(species_hbm.at[pl.ds(off, chunk)],
                                  s1_v, sems.at[0, b]),
            pltpu.make_async_copy(species_hbm.at[pl.ds(p_total + off, chunk)],
                                  s2_v, sems.at[1, b]),
            pltpu.make_async_copy(dist_hbm.at[pl.ds(off, chunk)],
                                  d_v, sems.at[2, b]),
        ]

    def out_copy(c, b):
        off = base + c * chunk
        return pltpu.make_async_copy(bufs[b][3],
                                     out_hbm.at[pl.ds(off, chunk)],
                                     sems.at[3, b])

    for cp in in_copies(0, 0):
        cp.start()

    for c in range(n_chunks):
        b = c % 2
        s1_v, s2_v, d_v, o_v = bufs[b]

        if c + 1 < n_chunks:
            for cp in in_copies(c + 1, (c + 1) % 2):
                cp.start()

        for cp in in_copies(c, b):
            cp.wait()
        if c >= 2:
            out_copy(c - 2, b).wait()

        def vec_body(i, _):
            sl = pl.ds(i * LANES, LANES)
            idx = s1_v[sl] * 4 + s2_v[sl]
            damp = lax.gather(
                damp_tbl, idx[:, None],
                lax.GatherDimensionNumbers(offset_dims=(),
                                           collapsed_slice_dims=(0,),
                                           start_index_map=(0,)),
                slice_sizes=(1,),
                mode=lax.GatherScatterMode.PROMISE_IN_BOUNDS)
            dv = d_v[sl]
            o_v[sl] = _pow6(dv) + damp
            return 0

        lax.fori_loop(0, chunk // LANES, vec_body, 0, unroll=5)
        out_copy(c, b).start()

    if n_chunks >= 2:
        out_copy(n_chunks - 2, (n_chunks - 2) % 2).wait()
    out_copy(n_chunks - 1, (n_chunks - 1) % 2).wait()


def kernel(species12, distances, cutoff_radii):
    P = distances.shape[0]
    n_workers = 32
    per_worker = P // n_workers
    assert per_worker * n_workers == P
    chunk = 10_000
    n_chunks = per_worker // chunk
    assert n_chunks * chunk == per_worker and chunk % LANES == 0

    table_flat = cutoff_radii.astype(jnp.float32).reshape(16)
    species_flat = species12.astype(jnp.int32).reshape(2 * P)

    mesh = plsc.VectorSubcoreMesh(core_axis_name="c", subcore_axis_name="s")
    run = pl.kernel(
        functools.partial(_tec_body, chunk, n_chunks, P),
        mesh=mesh,
        out_type=jax.ShapeDtypeStruct((P,), jnp.float32),
        scratch_types=[
            pltpu.VMEM((16,), jnp.float32),
            pltpu.VMEM((chunk,), jnp.int32),
            pltpu.VMEM((chunk,), jnp.int32),
            pltpu.VMEM((chunk,), jnp.int32),
            pltpu.VMEM((chunk,), jnp.int32),
            pltpu.VMEM((chunk,), jnp.float32),
            pltpu.VMEM((chunk,), jnp.float32),
            pltpu.VMEM((chunk,), jnp.float32),
            pltpu.VMEM((chunk,), jnp.float32),
            pltpu.SemaphoreType.DMA((4, 2)),
        ],
    )
    return run(species_flat, distances.astype(jnp.float32), table_flat)
